# Initial kernel scaffold; baseline (speedup 1.0000x reference)
import functools

import jax
import jax.numpy as jnp
from jax import lax
from jax.experimental import pallas as pl
from jax.experimental.pallas import tpu as pltpu
from jax.experimental.pallas import tpu_sc as plsc

N = 10000
D = 128
E = 320000
MAXN = 1.0 - 1e-5

NC = 2
NS = 16
CH = 128
NP = 10240
EP = 327680
HALF = EP // 2
TPE = EP // (NC * NS)
CPT = TPE // CH
NPT = NP // NS

RB = 400
GRID = N // RB

f32 = jnp.float32
i32 = jnp.int32


def _u_scale(n):
    nu = jnp.minimum(jnp.tanh(n), MAXN)
    nu = jnp.maximum(nu, 1e-15)
    at = 0.5 * jnp.log((1.0 + nu) / (1.0 - nu))
    return at / n


def _mobius_attention_head(u, W_ref, b_ref, av_ref, h_ref, al_ref):
    h = jnp.dot(u, W_ref[...], preferred_element_type=f32) + b_ref[...]
    h_ref[...] = h
    al_ref[0:1, :] = jnp.sum(h * av_ref[0:1, :], axis=1)[None, :]
    al_ref[1:2, :] = jnp.sum(h * av_ref[1:2, :], axis=1)[None, :]


def _tc_first_body(x_ref, W_ref, b_ref, av_ref, h_ref, al_ref):
    xb = x_ref[...]
    n = jnp.maximum(jnp.sqrt(jnp.sum(xb * xb, axis=1, keepdims=True)), 1e-15)
    u = xb * _u_scale(n)
    _mobius_attention_head(u, W_ref, b_ref, av_ref, h_ref, al_ref)


def _tc_mid_body(agg_ref, W_ref, b_ref, av_ref, h_ref, al_ref):
    g = jnp.maximum(agg_ref[0] + agg_ref[1], 0.0)
    n = jnp.maximum(jnp.sqrt(jnp.sum(g * g, axis=1, keepdims=True)), 1e-15)
    u = g * _u_scale(n)
    _mobius_attention_head(u, W_ref, b_ref, av_ref, h_ref, al_ref)


def _tc_final_body(agg_ref, y_ref):
    g = jnp.maximum(agg_ref[0] + agg_ref[1], 0.0)
    n = jnp.maximum(jnp.sqrt(jnp.sum(g * g, axis=1, keepdims=True)), 1e-15)
    nu = jnp.minimum(jnp.tanh(n), MAXN)
    y_ref[...] = g * (nu / n)


_mat_specs = [
    pl.BlockSpec((D, D), lambda i: (0, 0)),
    pl.BlockSpec((1, D), lambda i: (0, 0)),
    pl.BlockSpec((2, D), lambda i: (0, 0)),
]
_head_out_shape = [
    jax.ShapeDtypeStruct((N, D), f32),
    jax.ShapeDtypeStruct((2, N), f32),
]
_head_out_specs = [
    pl.BlockSpec((RB, D), lambda i: (i, 0)),
    pl.BlockSpec((2, RB), lambda i: (0, i)),
]


def _tc_first(x, W, b2, av):
    return pl.pallas_call(
        _tc_first_body,
        grid=(GRID,),
        in_specs=[pl.BlockSpec((RB, D), lambda i: (i, 0))] + _mat_specs,
        out_specs=_head_out_specs,
        out_shape=_head_out_shape,
    )(x, W, b2, av)


def _tc_mid(aggp, W, b2, av):
    return pl.pallas_call(
        _tc_mid_body,
        grid=(GRID,),
        in_specs=[pl.BlockSpec((NC, RB, D), lambda i: (0, i, 0))] + _mat_specs,
        out_specs=_head_out_specs,
        out_shape=_head_out_shape,
    )(aggp, W, b2, av)


def _tc_final(aggp):
    return pl.pallas_call(
        _tc_final_body,
        grid=(GRID,),
        in_specs=[pl.BlockSpec((NC, RB, D), lambda i: (0, i, 0))],
        out_specs=pl.BlockSpec((RB, D), lambda i: (i, 0)),
        out_shape=jax.ShapeDtypeStruct((N, D), f32),
    )(aggp)


def _sc_body(h_hbm, alpha_hbm, src_hbm, dst_hbm, out_hbm,
             asrc_v, adst_v, denom_v, srcbuf, dstbuf, wbuf,
             osrc, odst, wtmp, attnbuf, rows, sem, denom_sp, agg_sp):
    c = lax.axis_index("c")
    s = lax.axis_index("s")
    zeros16 = jnp.zeros((16,), f32)

    pltpu.sync_copy(alpha_hbm.at[0], asrc_v.at[pl.ds(0, N)])
    pltpu.sync_copy(alpha_hbm.at[1], adst_v.at[pl.ds(0, N)])

    def _zpad(i, carry):
        asrc_v[pl.ds(N + i * 16, 16)] = zeros16
        adst_v[pl.ds(N + i * 16, 16)] = zeros16
        return carry

    lax.fori_loop(0, (NP - N) // 16, _zpad, 0)

    def _zrows(i, carry):
        for k in range(D // 16):
            rows[0, i, pl.ds(k * 16, 16)] = zeros16
        return carry

    lax.fori_loop(0, CH, _zrows, 0)

    nbase = s * NPT
    for k in range(NPT // CH):
        pltpu.sync_copy(rows.at[0], agg_sp.at[pl.ds(nbase + k * CH, CH), :])
        pltpu.sync_copy(rows.at[0, 0], denom_sp.at[pl.ds(nbase + k * CH, CH)])
    plsc.subcore_barrier()

    def _cmax(i, m):
        return jnp.maximum(m, asrc_v[pl.ds(i * 16, 16)])

    m16 = lax.fori_loop(0, NP // 16, _cmax, jnp.full((16,), -3.4e38, f32))
    C16 = jnp.full((16,), jnp.max(m16), f32)

    def _edge_w(sb, db, wb):
        for k in range(CH // 16):
            sv = sb[pl.ds(k * 16, 16)]
            dv = db[pl.ds(k * 16, 16)]
            a_s = plsc.load_gather(asrc_v, [sv])
            a_d = plsc.load_gather(adst_v, [dv])
            t = a_s + a_d
            e = jnp.maximum(t, 0.2 * t)
            m0 = C16 + a_d
            M = jnp.maximum(m0, 0.2 * m0)
            wb[pl.ds(k * 16, 16)] = jnp.exp(e - M)

    base_my = c * HALF + s * TPE
    base_ot = (1 - c) * HALF + s * TPE

    def _scalar_my(j, carry):
        off = base_my + j * CH
        pltpu.sync_copy(src_hbm.at[pl.ds(off, CH)], srcbuf.at[j])
        pltpu.sync_copy(dst_hbm.at[pl.ds(off, CH)], dstbuf.at[j])
        _edge_w(srcbuf.at[j], dstbuf.at[j], wbuf.at[j])
        pltpu.sync_copy(wbuf.at[j], denom_sp.at[dstbuf.at[j]], add=True)
        return carry

    def _scalar_other(j, carry):
        off = base_ot + j * CH
        pltpu.sync_copy(src_hbm.at[pl.ds(off, CH)], osrc)
        pltpu.sync_copy(dst_hbm.at[pl.ds(off, CH)], odst)
        _edge_w(osrc, odst, wtmp)
        pltpu.sync_copy(wtmp, denom_sp.at[odst], add=True)
        return carry

    lax.fori_loop(0, CPT, _scalar_my, 0)
    lax.fori_loop(0, CPT, _scalar_other, 0)
    plsc.subcore_barrier()

    pltpu.sync_copy(denom_sp, denom_v)

    pltpu.async_copy(h_hbm.at[srcbuf.at[0]], rows.at[0], sem)

    def _rowchunk(jo, carry):
        for b in range(2):
            j = jo * 2 + b
            nb = (b + 1) % 2
            pltpu.make_async_copy(h_hbm.at[srcbuf.at[j]], rows.at[b], sem).wait()

            @pl.when(j + 1 < CPT)
            def _():
                pltpu.async_copy(h_hbm.at[srcbuf.at[j + 1]], rows.at[nb], sem)

            for k in range(CH // 16):
                wv = wbuf[j, pl.ds(k * 16, 16)]
                dv = dstbuf[j, pl.ds(k * 16, 16)]
                dn = plsc.load_gather(denom_v, [dv])
                attnbuf[pl.ds(k * 16, 16)] = wv / (dn + 1e-15)

            def _srow(r, carry2):
                av = plsc.load_gather(attnbuf, [jnp.full((16,), r, i32)])
                for k in range(D // 16):
                    sl = pl.ds(k * 16, 16)
                    rows[b, r, sl] = rows[b, r, sl] * av
                return carry2

            lax.fori_loop(0, CH, _srow, 0)
            pltpu.sync_copy(rows.at[b], agg_sp.at[dstbuf.at[j]], add=True)
        return carry

    lax.fori_loop(0, CPT // 2, _rowchunk, 0)
    plsc.subcore_barrier()

    for k in range(NPT // CH):
        sl = pl.ds(nbase + k * CH, CH)
        pltpu.sync_copy(agg_sp.at[sl, :], out_hbm.at[c, sl, :])


@functools.partial(
    pl.kernel,
    out_type=jax.ShapeDtypeStruct((NC, NP, D), f32),
    mesh=plsc.VectorSubcoreMesh(
        core_axis_name="c", subcore_axis_name="s", num_cores=NC, num_subcores=NS
    ),
    scratch_types=[
        pltpu.VMEM((NP,), f32),
        pltpu.VMEM((NP,), f32),
        pltpu.VMEM((NP,), f32),
        pltpu.VMEM((CPT, CH), i32),
        pltpu.VMEM((CPT, CH), i32),
        pltpu.VMEM((CPT, CH), f32),
        pltpu.VMEM((CH,), i32),
        pltpu.VMEM((CH,), i32),
        pltpu.VMEM((CH,), f32),
        pltpu.VMEM((CH,), f32),
        pltpu.VMEM((2, CH, D), f32),
        pltpu.SemaphoreType.DMA,
        pltpu.VMEM_SHARED((NP,), f32),
        pltpu.VMEM_SHARED((NP, D), f32),
    ],
)
def _sc_edge(h_hbm, alpha_hbm, src_hbm, dst_hbm, out_hbm,
             asrc_v, adst_v, denom_v, srcbuf, dstbuf, wbuf,
             osrc, odst, wtmp, attnbuf, rows, sem, denom_sp, agg_sp):
    _sc_body(h_hbm, alpha_hbm, src_hbm, dst_hbm, out_hbm,
             asrc_v, adst_v, denom_v, srcbuf, dstbuf, wbuf,
             osrc, odst, wtmp, attnbuf, rows, sem, denom_sp, agg_sp)


def kernel(x, edge_index, W1, b1, a_src1, a_dst1, W2, b2, a_src2, a_dst2):
    pad = EP - E
    srcp = jnp.concatenate([edge_index[0], jnp.zeros((pad,), i32)])
    dstp = jnp.concatenate([edge_index[1], jnp.full((pad,), NP - 1, i32)])

    av1 = jnp.stack([a_src1, a_dst1])
    av2 = jnp.stack([a_src2, a_dst2])

    h1, alpha1 = _tc_first(x, W1, b1[None, :], av1)
    aggp1 = _sc_edge(h1, alpha1, srcp, dstp)
    h2, alpha2 = _tc_mid(aggp1, W2, b2[None, :], av2)
    aggp2 = _sc_edge(h2, alpha2, srcp, dstp)
    return _tc_final(aggp2)


# speedup vs baseline: 9.3993x; 9.3993x over previous
import functools

import jax
import jax.numpy as jnp
from jax import lax
from jax.experimental import pallas as pl
from jax.experimental.pallas import tpu as pltpu
from jax.experimental.pallas import tpu_sc as plsc

N = 10000
D = 128
E = 320000
MAXN = 1.0 - 1e-5

NC = 2
NS = 16
CH = 128
DH = D // NC
NP = 10240
EP = 327680
TPE = EP // NS
CPT = TPE // CH
NPT = NP // NS

RBH = 512
GRIDH = NP // RBH
RB = 400
GRID = N // RB

f32 = jnp.float32
i32 = jnp.int32


def _u_scale(n):
    nu = jnp.minimum(jnp.tanh(n), MAXN)
    nu = jnp.maximum(nu, 1e-15)
    at = 0.5 * jnp.log((1.0 + nu) / (1.0 - nu))
    return at / n


def _mobius_attention_head(u, W_ref, b_ref, av_ref, h_ref, al_ref, cm_ref):
    h = jnp.dot(u, W_ref[...], preferred_element_type=f32) + b_ref[...]
    h_ref[0, :, :] = h[:, 0:DH]
    h_ref[1, :, :] = h[:, DH:D]
    i = pl.program_id(0)
    sl = pl.ds(i * RBH, RBH)
    asrc = jnp.sum(h * av_ref[0:1, :], axis=1)
    al_ref[0:1, sl] = asrc[None, :]
    al_ref[1:2, sl] = jnp.sum(h * av_ref[1:2, :], axis=1)[None, :]
    blkmax = jnp.full((1, D), jnp.max(asrc), f32)

    @pl.when(i == 0)
    def _():
        cm_ref[...] = blkmax

    cm_ref[...] = jnp.maximum(cm_ref[...], blkmax)


def _tc_first_body(x_ref, W_ref, b_ref, av_ref, h_ref, al_ref, cm_ref):
    xb = x_ref[...]
    n = jnp.maximum(jnp.sqrt(jnp.sum(xb * xb, axis=1, keepdims=True)), 1e-15)
    u = xb * _u_scale(n)
    _mobius_attention_head(u, W_ref, b_ref, av_ref, h_ref, al_ref, cm_ref)


def _tc_mid_body(agg_ref, W_ref, b_ref, av_ref, h_ref, al_ref, cm_ref):
    g = jnp.maximum(jnp.concatenate([agg_ref[0], agg_ref[1]], axis=1), 0.0)
    n = jnp.maximum(jnp.sqrt(jnp.sum(g * g, axis=1, keepdims=True)), 1e-15)
    u = g * _u_scale(n)
    _mobius_attention_head(u, W_ref, b_ref, av_ref, h_ref, al_ref, cm_ref)


def _tc_final_body(agg_ref, y_ref):
    g = jnp.maximum(jnp.concatenate([agg_ref[0], agg_ref[1]], axis=1), 0.0)
    n = jnp.maximum(jnp.sqrt(jnp.sum(g * g, axis=1, keepdims=True)), 1e-15)
    nu = jnp.minimum(jnp.tanh(n), MAXN)
    y_ref[...] = g * (nu / n)


_mat_specs = [
    pl.BlockSpec((D, D), lambda i: (0, 0)),
    pl.BlockSpec((1, D), lambda i: (0, 0)),
    pl.BlockSpec((2, D), lambda i: (0, 0)),
]
_head_out_shape = [
    jax.ShapeDtypeStruct((NC, NP, DH), f32),
    jax.ShapeDtypeStruct((2, NP), f32),
    jax.ShapeDtypeStruct((1, D), f32),
]
_head_out_specs = [
    pl.BlockSpec((NC, RBH, DH), lambda i: (0, i, 0)),
    pl.BlockSpec((2, NP), lambda i: (0, 0)),
    pl.BlockSpec((1, D), lambda i: (0, 0)),
]


def _tc_first(x, W, b2, av):
    return pl.pallas_call(
        _tc_first_body,
        grid=(GRIDH,),
        in_specs=[pl.BlockSpec((RBH, D), lambda i: (i, 0))] + _mat_specs,
        out_specs=_head_out_specs,
        out_shape=_head_out_shape,
    )(x, W, b2, av)


def _tc_mid(aggp, W, b2, av):
    return pl.pallas_call(
        _tc_mid_body,
        grid=(GRIDH,),
        in_specs=[pl.BlockSpec((NC, RBH, DH), lambda i: (0, i, 0))] + _mat_specs,
        out_specs=_head_out_specs,
        out_shape=_head_out_shape,
    )(aggp, W, b2, av)


def _tc_final(aggp):
    return pl.pallas_call(
        _tc_final_body,
        grid=(GRID,),
        in_specs=[pl.BlockSpec((NC, RB, DH), lambda i: (0, i, 0))],
        out_specs=pl.BlockSpec((RB, D), lambda i: (i, 0)),
        out_shape=jax.ShapeDtypeStruct((N, D), f32),
    )(aggp)


def _sc_body(h_hbm, alpha_hbm, cm_hbm, src_hbm, dst_hbm, out_hbm,
             asrc_v, adst_v, denom_v, srcbuf, dstbuf, wbuf,
             attnbuf, cbuf, zbuf, rows, sem, denom_sp, agg_sp):
    c = lax.axis_index("c")
    s = lax.axis_index("s")
    zeros16 = jnp.zeros((16,), f32)

    pltpu.sync_copy(cm_hbm.at[0], cbuf)

    pltpu.sync_copy(alpha_hbm.at[0], asrc_v)
    pltpu.sync_copy(alpha_hbm.at[1], adst_v)

    def _zrows(i, carry):
        for k in range(DH // 16):
            rows[i, pl.ds(k * 16, 16)] = zeros16
        return carry

    lax.fori_loop(0, CH, _zrows, 0)
    for k in range(CH // 16):
        zbuf[pl.ds(k * 16, 16)] = zeros16

    nbase = s * NPT
    for k in range(NPT // CH):
        pltpu.sync_copy(rows, agg_sp.at[pl.ds(nbase + k * CH, CH), :])
        pltpu.sync_copy(zbuf, denom_sp.at[pl.ds(nbase + k * CH, CH)])
    plsc.subcore_barrier()

    C16 = jnp.maximum(cbuf[pl.ds(0, 16)], 0.0)

    base_e = s * TPE

    def _edge_w(sb, db, k):
        sv = sb[pl.ds(k * 16, 16)]
        dv = db[pl.ds(k * 16, 16)]
        a_s = plsc.load_gather(asrc_v, [sv])
        a_d = plsc.load_gather(adst_v, [dv])
        t = a_s + a_d
        e = jnp.maximum(t, 0.2 * t)
        m0 = C16 + a_d
        M = jnp.maximum(m0, 0.2 * m0)
        return jnp.exp(e - M)

    def _scalar(j, carry):
        off = base_e + j * CH
        pltpu.sync_copy(src_hbm.at[pl.ds(off, CH)], srcbuf)
        pltpu.sync_copy(dst_hbm.at[pl.ds(off, CH)], dstbuf)
        for k in range(CH // 16):
            wbuf[pl.ds(k * 16, 16)] = _edge_w(srcbuf, dstbuf, k)
        pltpu.sync_copy(wbuf, denom_sp.at[dstbuf], add=True)
        return carry

    lax.fori_loop(0, CPT, _scalar, 0)
    plsc.subcore_barrier()

    pltpu.sync_copy(denom_sp, denom_v)

    hc = h_hbm.at[c]

    def _rowchunk(j, carry):
        off = base_e + j * CH
        pltpu.sync_copy(src_hbm.at[pl.ds(off, CH)], srcbuf)
        pltpu.sync_copy(dst_hbm.at[pl.ds(off, CH)], dstbuf)
        pltpu.async_copy(hc.at[srcbuf], rows, sem).wait()
        for k in range(CH // 16):
            wv = _edge_w(srcbuf, dstbuf, k)
            dv = dstbuf[pl.ds(k * 16, 16)]
            dn = plsc.load_gather(denom_v, [dv])
            attnbuf[pl.ds(k * 16, 16)] = wv / (dn + 1e-15)

        def _srow(r, carry2):
            av = plsc.load_gather(attnbuf, [jnp.full((16,), r, i32)])
            for k in range(DH // 16):
                sl = pl.ds(k * 16, 16)
                rows[r, sl] = rows[r, sl] * av
            return carry2

        lax.fori_loop(0, CH, _srow, 0)
        pltpu.sync_copy(rows, agg_sp.at[dstbuf], add=True)
        return carry

    lax.fori_loop(0, CPT, _rowchunk, 0)
    plsc.subcore_barrier()

    for k in range(NPT // CH):
        sl = pl.ds(nbase + k * CH, CH)
        pltpu.sync_copy(agg_sp.at[sl, :], out_hbm.at[c, sl, :])


@functools.cache
def _sc_edge_kernel():
    return pl.kernel(
        _sc_body,
        out_type=jax.ShapeDtypeStruct((NC, NP, DH), f32),
        mesh=plsc.VectorSubcoreMesh(
            core_axis_name="c", subcore_axis_name="s",
            num_cores=NC, num_subcores=NS,
        ),
        compiler_params=pltpu.CompilerParams(
            needs_layout_passes=False, use_tc_tiling_on_sc=False
        ),
        scratch_types=[
            pltpu.VMEM((NP,), f32),
            pltpu.VMEM((NP,), f32),
            pltpu.VMEM((NP,), f32),
            pltpu.VMEM((CH,), i32),
            pltpu.VMEM((CH,), i32),
            pltpu.VMEM((CH,), f32),
            pltpu.VMEM((CH,), f32),
            pltpu.VMEM((D,), f32),
            pltpu.VMEM((CH,), f32),
            pltpu.VMEM((CH, DH), f32),
            pltpu.SemaphoreType.DMA,
            pltpu.VMEM_SHARED((NP,), f32),
            pltpu.VMEM_SHARED((NP, DH), f32),
        ],
    )


def _sc_edge(h, alpha, cm, srcp, dstp):
    return _sc_edge_kernel()(h, alpha, cm, srcp, dstp)


def kernel(x, edge_index, W1, b1, a_src1, a_dst1, W2, b2, a_src2, a_dst2):
    pad = EP - E
    srcp = jnp.concatenate([edge_index[0], jnp.zeros((pad,), i32)])
    dstp = jnp.concatenate([edge_index[1], jnp.full((pad,), NP - 1, i32)])
    xp = jnp.concatenate([x, jnp.zeros((NP - N, D), f32)])

    av1 = jnp.stack([a_src1, a_dst1])
    av2 = jnp.stack([a_src2, a_dst2])

    h1, alpha1, cm1 = _tc_first(xp, W1, b1[None, :], av1)
    aggp1 = _sc_edge(h1, alpha1, cm1, srcp, dstp)
    h2, alpha2, cm2 = _tc_mid(aggp1, W2, b2[None, :], av2)
    aggp2 = _sc_edge(h2, alpha2, cm2, srcp, dstp)
    return _tc_final(aggp2)


# speedup vs baseline: 11.0435x; 1.1749x over previous
import functools

import jax
import jax.numpy as jnp
from jax import lax
from jax.experimental import pallas as pl
from jax.experimental.pallas import tpu as pltpu
from jax.experimental.pallas import tpu_sc as plsc

N = 10000
D = 128
E = 320000
MAXN = 1.0 - 1e-5

NC = 2
NS = 16
CH = 128
DH = D // NC
NP = 10240
EP = 327680
TPE = EP // NS
CPT = TPE // CH
NPT = NP // NS

RBH = 512
GRIDH = NP // RBH
RB = 400
GRID = N // RB

f32 = jnp.float32
i32 = jnp.int32


def _u_scale(n):
    nu = jnp.minimum(jnp.tanh(n), MAXN)
    nu = jnp.maximum(nu, 1e-15)
    at = 0.5 * jnp.log((1.0 + nu) / (1.0 - nu))
    return at / n


def _mobius_attention_head(u, W_ref, b_ref, av_ref, h_ref, al_ref, cm_ref):
    h = jnp.dot(u, W_ref[...], preferred_element_type=f32) + b_ref[...]
    h_ref[0, :, :] = h[:, 0:DH]
    h_ref[1, :, :] = h[:, DH:D]
    i = pl.program_id(0)
    sl = pl.ds(i * RBH, RBH)
    asrc = jnp.sum(h * av_ref[0:1, :], axis=1)
    al_ref[0:1, sl] = asrc[None, :]
    al_ref[1:2, sl] = jnp.sum(h * av_ref[1:2, :], axis=1)[None, :]
    blkmax = jnp.full((1, D), jnp.max(asrc), f32)

    @pl.when(i == 0)
    def _():
        cm_ref[...] = blkmax

    cm_ref[...] = jnp.maximum(cm_ref[...], blkmax)


def _tc_first_body(x_ref, W_ref, b_ref, av_ref, h_ref, al_ref, cm_ref):
    xb = x_ref[...]
    n = jnp.maximum(jnp.sqrt(jnp.sum(xb * xb, axis=1, keepdims=True)), 1e-15)
    u = xb * _u_scale(n)
    _mobius_attention_head(u, W_ref, b_ref, av_ref, h_ref, al_ref, cm_ref)


def _tc_mid_body(agg_ref, W_ref, b_ref, av_ref, h_ref, al_ref, cm_ref):
    g = jnp.maximum(jnp.concatenate([agg_ref[0], agg_ref[1]], axis=1), 0.0)
    n = jnp.maximum(jnp.sqrt(jnp.sum(g * g, axis=1, keepdims=True)), 1e-15)
    u = g * _u_scale(n)
    _mobius_attention_head(u, W_ref, b_ref, av_ref, h_ref, al_ref, cm_ref)


def _tc_final_body(agg_ref, y_ref):
    g = jnp.maximum(jnp.concatenate([agg_ref[0], agg_ref[1]], axis=1), 0.0)
    n = jnp.maximum(jnp.sqrt(jnp.sum(g * g, axis=1, keepdims=True)), 1e-15)
    nu = jnp.minimum(jnp.tanh(n), MAXN)
    y_ref[...] = g * (nu / n)


_mat_specs = [
    pl.BlockSpec((D, D), lambda i: (0, 0)),
    pl.BlockSpec((1, D), lambda i: (0, 0)),
    pl.BlockSpec((2, D), lambda i: (0, 0)),
]
_head_out_shape = [
    jax.ShapeDtypeStruct((NC, NP, DH), f32),
    jax.ShapeDtypeStruct((2, NP), f32),
    jax.ShapeDtypeStruct((1, D), f32),
]
_head_out_specs = [
    pl.BlockSpec((NC, RBH, DH), lambda i: (0, i, 0)),
    pl.BlockSpec((2, NP), lambda i: (0, 0)),
    pl.BlockSpec((1, D), lambda i: (0, 0)),
]


def _tc_first(x, W, b2, av):
    return pl.pallas_call(
        _tc_first_body,
        grid=(GRIDH,),
        in_specs=[pl.BlockSpec((RBH, D), lambda i: (i, 0))] + _mat_specs,
        out_specs=_head_out_specs,
        out_shape=_head_out_shape,
    )(x, W, b2, av)


def _tc_mid(aggp, W, b2, av):
    return pl.pallas_call(
        _tc_mid_body,
        grid=(GRIDH,),
        in_specs=[pl.BlockSpec((NC, RBH, DH), lambda i: (0, i, 0))] + _mat_specs,
        out_specs=_head_out_specs,
        out_shape=_head_out_shape,
    )(aggp, W, b2, av)


def _tc_final(aggp):
    return pl.pallas_call(
        _tc_final_body,
        grid=(GRID,),
        in_specs=[pl.BlockSpec((NC, RB, DH), lambda i: (0, i, 0))],
        out_specs=pl.BlockSpec((RB, D), lambda i: (i, 0)),
        out_shape=jax.ShapeDtypeStruct((N, D), f32),
    )(aggp)


def _sc_body(h_hbm, alpha_hbm, cm_hbm, edge_hbm, out_hbm,
             asrc_v, adst_v, denom_v, ebuf, wbig,
             attnbuf, cbuf, zbuf, rows, sem, denom_sp, agg_sp):
    c = lax.axis_index("c")
    s = lax.axis_index("s")
    zeros16 = jnp.zeros((16,), f32)

    pltpu.sync_copy(cm_hbm.at[0], cbuf)

    pltpu.sync_copy(alpha_hbm.at[0], asrc_v)
    pltpu.sync_copy(alpha_hbm.at[1], adst_v)

    def _zrows(i, carry):
        for k in range(DH // 16):
            rows[i, pl.ds(k * 16, 16)] = zeros16
        return carry

    lax.fori_loop(0, CH, _zrows, 0)
    for k in range(CH // 16):
        zbuf[pl.ds(k * 16, 16)] = zeros16

    nbase = s * NPT
    for k in range(NPT // CH):
        pltpu.sync_copy(rows, agg_sp.at[pl.ds(nbase + k * CH, CH), :])
        pltpu.sync_copy(zbuf, denom_sp.at[pl.ds(nbase + k * CH, CH)])
    plsc.subcore_barrier()

    C16 = jnp.maximum(cbuf[pl.ds(0, 16)], 0.0)

    base_e = s * TPE

    def _edge_w(sb, db, k):
        sv = sb[pl.ds(k * 16, 16)]
        dv = db[pl.ds(k * 16, 16)]
        a_s = plsc.load_gather(asrc_v, [sv])
        a_d = plsc.load_gather(adst_v, [dv])
        t = a_s + a_d
        e = jnp.maximum(t, 0.2 * t)
        m0 = C16 + a_d
        M = jnp.maximum(m0, 0.2 * m0)
        return jnp.exp(e - M)

    def _scalar(j, carry):
        off = base_e + j * CH
        pltpu.sync_copy(edge_hbm.at[:, pl.ds(off, CH)], ebuf)
        sb, db, wb = ebuf.at[0], ebuf.at[1], wbig.at[j]
        for k in range(CH // 16):
            wb[pl.ds(k * 16, 16)] = _edge_w(sb, db, k)
        pltpu.sync_copy(wb, denom_sp.at[db], add=True)
        return carry

    lax.fori_loop(0, CPT, _scalar, 0)
    plsc.subcore_barrier()

    pltpu.sync_copy(denom_sp, denom_v)

    hc = h_hbm.at[c]

    def _rowchunk(j, carry):
        off = base_e + j * CH
        pltpu.sync_copy(edge_hbm.at[:, pl.ds(off, CH)], ebuf)
        sb, db = ebuf.at[0], ebuf.at[1]
        pltpu.async_copy(hc.at[sb], rows, sem).wait()
        for k in range(CH // 16):
            wv = wbig[j, pl.ds(k * 16, 16)]
            dv = db[pl.ds(k * 16, 16)]
            dn = plsc.load_gather(denom_v, [dv])
            attnbuf[pl.ds(k * 16, 16)] = wv / (dn + 1e-15)

        def _srow(ro, carry2):
            for u in range(2):
                r = ro * 2 + u
                av = plsc.load_gather(attnbuf, [jnp.full((16,), r, i32)])
                for k in range(DH // 16):
                    sl = pl.ds(k * 16, 16)
                    rows[r, sl] = rows[r, sl] * av
            return carry2

        lax.fori_loop(0, CH // 2, _srow, 0)
        pltpu.sync_copy(rows, agg_sp.at[db], add=True)
        return carry

    lax.fori_loop(0, CPT, _rowchunk, 0)
    plsc.subcore_barrier()

    for k in range(NPT // CH):
        sl = pl.ds(nbase + k * CH, CH)
        pltpu.sync_copy(agg_sp.at[sl, :], out_hbm.at[c, sl, :])


@functools.cache
def _sc_edge_kernel():
    return pl.kernel(
        _sc_body,
        out_type=jax.ShapeDtypeStruct((NC, NP, DH), f32),
        mesh=plsc.VectorSubcoreMesh(
            core_axis_name="c", subcore_axis_name="s",
            num_cores=NC, num_subcores=NS,
        ),
        compiler_params=pltpu.CompilerParams(
            needs_layout_passes=False, use_tc_tiling_on_sc=False
        ),
        scratch_types=[
            pltpu.VMEM((NP,), f32),
            pltpu.VMEM((NP,), f32),
            pltpu.VMEM((NP,), f32),
            pltpu.VMEM((2, CH), i32),
            pltpu.VMEM((CPT, CH), f32),
            pltpu.VMEM((CH,), f32),
            pltpu.VMEM((D,), f32),
            pltpu.VMEM((CH,), f32),
            pltpu.VMEM((CH, DH), f32),
            pltpu.SemaphoreType.DMA,
            pltpu.VMEM_SHARED((NP,), f32),
            pltpu.VMEM_SHARED((NP, DH), f32),
        ],
    )


def _sc_edge(h, alpha, cm, edges):
    return _sc_edge_kernel()(h, alpha, cm, edges)


def kernel(x, edge_index, W1, b1, a_src1, a_dst1, W2, b2, a_src2, a_dst2):
    pad = EP - E
    srcp = jnp.concatenate([edge_index[0], jnp.zeros((pad,), i32)])
    dstp = jnp.concatenate([edge_index[1], jnp.full((pad,), NP - 1, i32)])
    edges = jnp.stack([srcp, dstp])
    xp = jnp.concatenate([x, jnp.zeros((NP - N, D), f32)])

    av1 = jnp.stack([a_src1, a_dst1])
    av2 = jnp.stack([a_src2, a_dst2])

    h1, alpha1, cm1 = _tc_first(xp, W1, b1[None, :], av1)
    aggp1 = _sc_edge(h1, alpha1, cm1, edges)
    h2, alpha2, cm2 = _tc_mid(aggp1, W2, b2[None, :], av2)
    aggp2 = _sc_edge(h2, alpha2, cm2, edges)
    return _tc_final(aggp2)


# speedup vs baseline: 13.9211x; 1.2606x over previous
import functools

import jax
import jax.numpy as jnp
from jax import lax
from jax.experimental import pallas as pl
from jax.experimental.pallas import tpu as pltpu
from jax.experimental.pallas import tpu_sc as plsc

N = 10000
D = 128
E = 320000
MAXN = 1.0 - 1e-5

NC = 2
NS = 16
CH = 128
DH = D // NC
NP = 10240
EP = 327680
TPE = EP // NS
CPT = TPE // CH
NPT = NP // NS

RBH = 512
GRIDH = NP // RBH
RB = 400
GRID = N // RB

f32 = jnp.float32
i32 = jnp.int32


def _u_scale(n):
    nu = jnp.minimum(jnp.tanh(n), MAXN)
    nu = jnp.maximum(nu, 1e-15)
    at = 0.5 * jnp.log((1.0 + nu) / (1.0 - nu))
    return at / n


def _mobius_attention_head(u, W_ref, b_ref, av_ref, h_ref, al_ref, cm_ref):
    h = jnp.dot(u, W_ref[...], preferred_element_type=f32) + b_ref[...]
    h_ref[0, :, :] = h[:, 0:DH]
    h_ref[1, :, :] = h[:, DH:D]
    i = pl.program_id(0)
    sl = pl.ds(i * RBH, RBH)
    asrc = jnp.sum(h * av_ref[0:1, :], axis=1)
    al_ref[0:1, sl] = asrc[None, :]
    al_ref[1:2, sl] = jnp.sum(h * av_ref[1:2, :], axis=1)[None, :]
    blkmax = jnp.full((1, D), jnp.max(asrc), f32)

    @pl.when(i == 0)
    def _():
        cm_ref[...] = blkmax

    cm_ref[...] = jnp.maximum(cm_ref[...], blkmax)


def _tc_first_body(x_ref, W_ref, b_ref, av_ref, h_ref, al_ref, cm_ref):
    xb = x_ref[...]
    n = jnp.maximum(jnp.sqrt(jnp.sum(xb * xb, axis=1, keepdims=True)), 1e-15)
    u = xb * _u_scale(n)
    _mobius_attention_head(u, W_ref, b_ref, av_ref, h_ref, al_ref, cm_ref)


def _tc_mid_body(agg_ref, W_ref, b_ref, av_ref, h_ref, al_ref, cm_ref):
    g = jnp.maximum(jnp.concatenate([agg_ref[0], agg_ref[1]], axis=1), 0.0)
    n = jnp.maximum(jnp.sqrt(jnp.sum(g * g, axis=1, keepdims=True)), 1e-15)
    u = g * _u_scale(n)
    _mobius_attention_head(u, W_ref, b_ref, av_ref, h_ref, al_ref, cm_ref)


def _tc_final_body(agg_ref, y_ref):
    g = jnp.maximum(jnp.concatenate([agg_ref[0], agg_ref[1]], axis=1), 0.0)
    n = jnp.maximum(jnp.sqrt(jnp.sum(g * g, axis=1, keepdims=True)), 1e-15)
    nu = jnp.minimum(jnp.tanh(n), MAXN)
    y_ref[...] = g * (nu / n)


_mat_specs = [
    pl.BlockSpec((D, D), lambda i: (0, 0)),
    pl.BlockSpec((1, D), lambda i: (0, 0)),
    pl.BlockSpec((2, D), lambda i: (0, 0)),
]
_head_out_shape = [
    jax.ShapeDtypeStruct((NC, NP, DH), f32),
    jax.ShapeDtypeStruct((2, NP), f32),
    jax.ShapeDtypeStruct((1, D), f32),
]
_head_out_specs = [
    pl.BlockSpec((NC, RBH, DH), lambda i: (0, i, 0)),
    pl.BlockSpec((2, NP), lambda i: (0, 0)),
    pl.BlockSpec((1, D), lambda i: (0, 0)),
]


def _tc_first(x, W, b2, av):
    return pl.pallas_call(
        _tc_first_body,
        grid=(GRIDH,),
        in_specs=[pl.BlockSpec((RBH, D), lambda i: (i, 0))] + _mat_specs,
        out_specs=_head_out_specs,
        out_shape=_head_out_shape,
    )(x, W, b2, av)


def _tc_mid(aggp, W, b2, av):
    return pl.pallas_call(
        _tc_mid_body,
        grid=(GRIDH,),
        in_specs=[pl.BlockSpec((NC, RBH, DH), lambda i: (0, i, 0))] + _mat_specs,
        out_specs=_head_out_specs,
        out_shape=_head_out_shape,
    )(aggp, W, b2, av)


def _tc_final(aggp):
    return pl.pallas_call(
        _tc_final_body,
        grid=(GRID,),
        in_specs=[pl.BlockSpec((NC, RB, DH), lambda i: (0, i, 0))],
        out_specs=pl.BlockSpec((RB, D), lambda i: (i, 0)),
        out_shape=jax.ShapeDtypeStruct((N, D), f32),
    )(aggp)


def _sc_body(h_hbm, alpha_hbm, cm_hbm, edge_hbm, out_hbm,
             asrc_v, adst_v, denom_v, ebuf, rbuf, wbig,
             attnbuf, cbuf, zbuf, rows, sem, denom_sp, agg_sp):
    c = lax.axis_index("c")
    s = lax.axis_index("s")
    zeros16 = jnp.zeros((16,), f32)

    pltpu.sync_copy(cm_hbm.at[0], cbuf)

    pltpu.sync_copy(alpha_hbm.at[0], asrc_v)
    pltpu.sync_copy(alpha_hbm.at[1], adst_v)

    def _zrows(i, carry):
        for k in range(DH // 16):
            rows[0, i, pl.ds(k * 16, 16)] = zeros16
        return carry

    lax.fori_loop(0, CH, _zrows, 0)
    for k in range(CH // 16):
        zbuf[pl.ds(k * 16, 16)] = zeros16

    nbase = s * NPT
    for k in range(NPT // CH):
        pltpu.sync_copy(rows.at[0], agg_sp.at[pl.ds(nbase + k * CH, CH), :])
        pltpu.sync_copy(zbuf, denom_sp.at[pl.ds(nbase + k * CH, CH)])
    plsc.subcore_barrier()

    C16 = jnp.maximum(cbuf[pl.ds(0, 16)], 0.0)

    base_e = s * TPE

    def _edge_w(sb, db, k):
        sv = sb[pl.ds(k * 16, 16)]
        dv = db[pl.ds(k * 16, 16)]
        a_s = plsc.load_gather(asrc_v, [sv])
        a_d = plsc.load_gather(adst_v, [dv])
        t = a_s + a_d
        e = jnp.maximum(t, 0.2 * t)
        m0 = C16 + a_d
        M = jnp.maximum(m0, 0.2 * m0)
        return jnp.exp(e - M)

    def _scalar(j, carry):
        off = base_e + j * CH
        pltpu.sync_copy(edge_hbm.at[:, pl.ds(off, CH)], ebuf)
        sb, db, wb = ebuf.at[0], ebuf.at[1], wbig.at[j]
        for k in range(CH // 16):
            wb[pl.ds(k * 16, 16)] = _edge_w(sb, db, k)
        pltpu.sync_copy(wb, denom_sp.at[db], add=True)
        return carry

    lax.fori_loop(0, CPT, _scalar, 0)
    plsc.subcore_barrier()

    pltpu.sync_copy(denom_sp, denom_v)

    hc = h_hbm.at[c]

    def _stage(j, b):
        off = base_e + j * CH
        pltpu.sync_copy(edge_hbm.at[:, pl.ds(off, CH)], rbuf.at[b])

    _stage(0, 0)
    pltpu.async_copy(hc.at[rbuf.at[0, 0]], rows.at[0], sem)

    def _rowchunk(jo, carry):
        for b in range(2):
            j = jo * 2 + b
            nb = (b + 1) % 2
            sb, db = rbuf.at[b, 0], rbuf.at[b, 1]
            pltpu.make_async_copy(hc.at[sb], rows.at[b], sem).wait()

            @pl.when(j + 1 < CPT)
            def _():
                _stage(j + 1, nb)
                pltpu.async_copy(hc.at[rbuf.at[nb, 0]], rows.at[nb], sem)

            for k in range(CH // 16):
                wv = wbig[j, pl.ds(k * 16, 16)]
                dv = db[pl.ds(k * 16, 16)]
                dn = plsc.load_gather(denom_v, [dv])
                attnbuf[pl.ds(k * 16, 16)] = wv / (dn + 1e-15)

            def _srow(ro, carry2):
                for u in range(2):
                    r = ro * 2 + u
                    av = plsc.load_gather(attnbuf, [jnp.full((16,), r, i32)])
                    for k in range(DH // 16):
                        sl = pl.ds(k * 16, 16)
                        rows[b, r, sl] = rows[b, r, sl] * av
                return carry2

            lax.fori_loop(0, CH // 2, _srow, 0)
            pltpu.sync_copy(rows.at[b], agg_sp.at[db], add=True)
        return carry

    lax.fori_loop(0, CPT // 2, _rowchunk, 0)
    plsc.subcore_barrier()

    for k in range(NPT // CH):
        sl = pl.ds(nbase + k * CH, CH)
        pltpu.sync_copy(agg_sp.at[sl, :], out_hbm.at[c, sl, :])


@functools.cache
def _sc_edge_kernel():
    return pl.kernel(
        _sc_body,
        out_type=jax.ShapeDtypeStruct((NC, NP, DH), f32),
        mesh=plsc.VectorSubcoreMesh(
            core_axis_name="c", subcore_axis_name="s",
            num_cores=NC, num_subcores=NS,
        ),
        compiler_params=pltpu.CompilerParams(
            needs_layout_passes=False, use_tc_tiling_on_sc=False
        ),
        scratch_types=[
            pltpu.VMEM((NP,), f32),
            pltpu.VMEM((NP,), f32),
            pltpu.VMEM((NP,), f32),
            pltpu.VMEM((2, CH), i32),
            pltpu.VMEM((2, 2, CH), i32),
            pltpu.VMEM((CPT, CH), f32),
            pltpu.VMEM((CH,), f32),
            pltpu.VMEM((D,), f32),
            pltpu.VMEM((CH,), f32),
            pltpu.VMEM((2, CH, DH), f32),
            pltpu.SemaphoreType.DMA,
            pltpu.VMEM_SHARED((NP,), f32),
            pltpu.VMEM_SHARED((NP, DH), f32),
        ],
    )


def _sc_edge(h, alpha, cm, edges):
    return _sc_edge_kernel()(h, alpha, cm, edges)


def kernel(x, edge_index, W1, b1, a_src1, a_dst1, W2, b2, a_src2, a_dst2):
    pad = EP - E
    srcp = jnp.concatenate([edge_index[0], jnp.zeros((pad,), i32)])
    dstp = jnp.concatenate([edge_index[1], jnp.full((pad,), NP - 1, i32)])
    edges = jnp.stack([srcp, dstp])
    xp = jnp.concatenate([x, jnp.zeros((NP - N, D), f32)])

    av1 = jnp.stack([a_src1, a_dst1])
    av2 = jnp.stack([a_src2, a_dst2])

    h1, alpha1, cm1 = _tc_first(xp, W1, b1[None, :], av1)
    aggp1 = _sc_edge(h1, alpha1, cm1, edges)
    h2, alpha2, cm2 = _tc_mid(aggp1, W2, b2[None, :], av2)
    aggp2 = _sc_edge(h2, alpha2, cm2, edges)
    return _tc_final(aggp2)


# speedup vs baseline: 13.9564x; 1.0025x over previous
import functools

import jax
import jax.numpy as jnp
from jax import lax
from jax.experimental import pallas as pl
from jax.experimental.pallas import tpu as pltpu
from jax.experimental.pallas import tpu_sc as plsc

N = 10000
D = 128
E = 320000
MAXN = 1.0 - 1e-5

NC = 2
NS = 16
CH = 128
DH = D // NC
NP = 10240
EP = 327680
TPE = EP // NS
CPT = TPE // CH
NPT = NP // NS

RBH = 512
GRIDH = NP // RBH
RB = 400
GRID = N // RB

f32 = jnp.float32
i32 = jnp.int32


def _u_scale(n):
    nu = jnp.minimum(jnp.tanh(n), MAXN)
    nu = jnp.maximum(nu, 1e-15)
    at = 0.5 * jnp.log((1.0 + nu) / (1.0 - nu))
    return at / n


def _mobius_attention_head(u, W_ref, b_ref, av_ref, h_ref, al_ref, cm_ref):
    h = jnp.dot(u, W_ref[...], preferred_element_type=f32) + b_ref[...]
    h_ref[0, :, :] = h[:, 0:DH]
    h_ref[1, :, :] = h[:, DH:D]
    i = pl.program_id(0)
    sl = pl.ds(i * RBH, RBH)
    asrc = jnp.sum(h * av_ref[0:1, :], axis=1)
    al_ref[0:1, sl] = asrc[None, :]
    al_ref[1:2, sl] = jnp.sum(h * av_ref[1:2, :], axis=1)[None, :]
    blkmax = jnp.full((1, D), jnp.max(asrc), f32)

    @pl.when(i == 0)
    def _():
        cm_ref[...] = blkmax

    cm_ref[...] = jnp.maximum(cm_ref[...], blkmax)


def _tc_first_body(x_ref, W_ref, b_ref, av_ref, h_ref, al_ref, cm_ref):
    xb = x_ref[...]
    n = jnp.maximum(jnp.sqrt(jnp.sum(xb * xb, axis=1, keepdims=True)), 1e-15)
    u = xb * _u_scale(n)
    _mobius_attention_head(u, W_ref, b_ref, av_ref, h_ref, al_ref, cm_ref)


def _tc_mid_body(agg_ref, W_ref, b_ref, av_ref, h_ref, al_ref, cm_ref):
    g = jnp.maximum(jnp.concatenate([agg_ref[0], agg_ref[1]], axis=1), 0.0)
    n = jnp.maximum(jnp.sqrt(jnp.sum(g * g, axis=1, keepdims=True)), 1e-15)
    u = g * _u_scale(n)
    _mobius_attention_head(u, W_ref, b_ref, av_ref, h_ref, al_ref, cm_ref)


def _tc_final_body(agg_ref, y_ref):
    g = jnp.maximum(jnp.concatenate([agg_ref[0], agg_ref[1]], axis=1), 0.0)
    n = jnp.maximum(jnp.sqrt(jnp.sum(g * g, axis=1, keepdims=True)), 1e-15)
    nu = jnp.minimum(jnp.tanh(n), MAXN)
    y_ref[...] = g * (nu / n)


_mat_specs = [
    pl.BlockSpec((D, D), lambda i: (0, 0)),
    pl.BlockSpec((1, D), lambda i: (0, 0)),
    pl.BlockSpec((2, D), lambda i: (0, 0)),
]
_head_out_shape = [
    jax.ShapeDtypeStruct((NC, NP, DH), f32),
    jax.ShapeDtypeStruct((2, NP), f32),
    jax.ShapeDtypeStruct((1, D), f32),
]
_head_out_specs = [
    pl.BlockSpec((NC, RBH, DH), lambda i: (0, i, 0)),
    pl.BlockSpec((2, NP), lambda i: (0, 0)),
    pl.BlockSpec((1, D), lambda i: (0, 0)),
]


def _tc_first(x, W, b2, av):
    return pl.pallas_call(
        _tc_first_body,
        grid=(GRIDH,),
        in_specs=[pl.BlockSpec((RBH, D), lambda i: (i, 0))] + _mat_specs,
        out_specs=_head_out_specs,
        out_shape=_head_out_shape,
    )(x, W, b2, av)


def _tc_mid(aggp, W, b2, av):
    return pl.pallas_call(
        _tc_mid_body,
        grid=(GRIDH,),
        in_specs=[pl.BlockSpec((NC, RBH, DH), lambda i: (0, i, 0))] + _mat_specs,
        out_specs=_head_out_specs,
        out_shape=_head_out_shape,
    )(aggp, W, b2, av)


def _tc_final(aggp):
    return pl.pallas_call(
        _tc_final_body,
        grid=(GRID,),
        in_specs=[pl.BlockSpec((NC, RB, DH), lambda i: (0, i, 0))],
        out_specs=pl.BlockSpec((RB, D), lambda i: (i, 0)),
        out_shape=jax.ShapeDtypeStruct((N, D), f32),
    )(aggp)


def _sc_body(h_hbm, alpha_hbm, cm_hbm, edge_hbm, out_hbm,
             asrc_v, adst_v, denom_v, ebuf, rbuf, wbig,
             attnbuf, cbuf, zbuf, rows, sem, sem2, denom_sp, agg_sp):
    c = lax.axis_index("c")
    s = lax.axis_index("s")
    zeros16 = jnp.zeros((16,), f32)

    pltpu.sync_copy(cm_hbm.at[0], cbuf)

    pltpu.sync_copy(alpha_hbm.at[0], asrc_v)
    pltpu.sync_copy(alpha_hbm.at[1], adst_v)

    def _zrows(i, carry):
        for k in range(DH // 16):
            rows[0, i, pl.ds(k * 16, 16)] = zeros16
        return carry

    lax.fori_loop(0, CH, _zrows, 0)
    for k in range(CH // 16):
        zbuf[pl.ds(k * 16, 16)] = zeros16

    nbase = s * NPT
    for k in range(NPT // CH):
        pltpu.sync_copy(rows.at[0], agg_sp.at[pl.ds(nbase + k * CH, CH), :])
        pltpu.sync_copy(zbuf, denom_sp.at[pl.ds(nbase + k * CH, CH)])
    plsc.subcore_barrier()

    C16 = jnp.maximum(cbuf[pl.ds(0, 16)], 0.0)

    base_e = s * TPE

    def _edge_w(sb, db, k):
        sv = sb[pl.ds(k * 16, 16)]
        dv = db[pl.ds(k * 16, 16)]
        a_s = plsc.load_gather(asrc_v, [sv])
        a_d = plsc.load_gather(adst_v, [dv])
        t = a_s + a_d
        e = jnp.maximum(t, 0.2 * t)
        m0 = C16 + a_d
        M = jnp.maximum(m0, 0.2 * m0)
        return jnp.exp(e - M)

    def _scalar(j, carry):
        off = base_e + j * CH
        pltpu.sync_copy(edge_hbm.at[:, pl.ds(off, CH)], ebuf)
        sb, db, wb = ebuf.at[0], ebuf.at[1], wbig.at[j]
        for k in range(CH // 16):
            wb[pl.ds(k * 16, 16)] = _edge_w(sb, db, k)
        pltpu.sync_copy(wb, denom_sp.at[db], add=True)
        return carry

    lax.fori_loop(0, CPT, _scalar, 0)
    plsc.subcore_barrier()

    pltpu.sync_copy(denom_sp, denom_v)

    hc = h_hbm.at[c]

    def _stage(j, b):
        off = base_e + j * CH
        pltpu.sync_copy(edge_hbm.at[:, pl.ds(off, CH)], rbuf.at[b])

    _stage(0, 0)
    pltpu.async_copy(hc.at[rbuf.at[0, 0]], rows.at[0], sem)

    def _rowchunk(jo, carry):
        for b in range(2):
            j = jo * 2 + b
            nb = (b + 1) % 2
            sb, db = rbuf.at[b, 0], rbuf.at[b, 1]
            pltpu.make_async_copy(hc.at[sb], rows.at[b], sem).wait()

            @pl.when(j + 1 < CPT)
            def _():
                @pl.when(j >= 1)
                def _():
                    pltpu.make_async_copy(
                        rows.at[nb], agg_sp.at[rbuf.at[nb, 1]], sem2
                    ).wait()

                _stage(j + 1, nb)
                pltpu.async_copy(hc.at[rbuf.at[nb, 0]], rows.at[nb], sem)

            for k in range(CH // 16):
                wv = wbig[j, pl.ds(k * 16, 16)]
                dv = db[pl.ds(k * 16, 16)]
                dn = plsc.load_gather(denom_v, [dv])
                attnbuf[pl.ds(k * 16, 16)] = wv / (dn + 1e-15)

            def _srow(ro, carry2):
                for u in range(4):
                    r = ro * 4 + u
                    av = plsc.load_gather(attnbuf, [jnp.full((16,), r, i32)])
                    for k in range(DH // 16):
                        sl = pl.ds(k * 16, 16)
                        rows[b, r, sl] = rows[b, r, sl] * av
                return carry2

            lax.fori_loop(0, CH // 4, _srow, 0)
            pltpu.async_copy(rows.at[b], agg_sp.at[db], add=True, sem=sem2)
        return carry

    lax.fori_loop(0, CPT // 2, _rowchunk, 0)
    pltpu.make_async_copy(
        rows.at[(CPT - 2) % 2], agg_sp.at[rbuf.at[(CPT - 2) % 2, 1]], sem2
    ).wait()
    pltpu.make_async_copy(
        rows.at[(CPT - 1) % 2], agg_sp.at[rbuf.at[(CPT - 1) % 2, 1]], sem2
    ).wait()
    plsc.subcore_barrier()

    for k in range(NPT // CH):
        sl = pl.ds(nbase + k * CH, CH)
        pltpu.sync_copy(agg_sp.at[sl, :], out_hbm.at[c, sl, :])


@functools.cache
def _sc_edge_kernel():
    return pl.kernel(
        _sc_body,
        out_type=jax.ShapeDtypeStruct((NC, NP, DH), f32),
        mesh=plsc.VectorSubcoreMesh(
            core_axis_name="c", subcore_axis_name="s",
            num_cores=NC, num_subcores=NS,
        ),
        compiler_params=pltpu.CompilerParams(
            needs_layout_passes=False, use_tc_tiling_on_sc=False
        ),
        scratch_types=[
            pltpu.VMEM((NP,), f32),
            pltpu.VMEM((NP,), f32),
            pltpu.VMEM((NP,), f32),
            pltpu.VMEM((2, CH), i32),
            pltpu.VMEM((2, 2, CH), i32),
            pltpu.VMEM((CPT, CH), f32),
            pltpu.VMEM((CH,), f32),
            pltpu.VMEM((D,), f32),
            pltpu.VMEM((CH,), f32),
            pltpu.VMEM((2, CH, DH), f32),
            pltpu.SemaphoreType.DMA,
            pltpu.SemaphoreType.DMA,
            pltpu.VMEM_SHARED((NP,), f32),
            pltpu.VMEM_SHARED((NP, DH), f32),
        ],
    )


def _sc_edge(h, alpha, cm, edges):
    return _sc_edge_kernel()(h, alpha, cm, edges)


def kernel(x, edge_index, W1, b1, a_src1, a_dst1, W2, b2, a_src2, a_dst2):
    pad = EP - E
    srcp = jnp.concatenate([edge_index[0], jnp.zeros((pad,), i32)])
    dstp = jnp.concatenate([edge_index[1], jnp.full((pad,), NP - 1, i32)])
    edges = jnp.stack([srcp, dstp])
    xp = jnp.concatenate([x, jnp.zeros((NP - N, D), f32)])

    av1 = jnp.stack([a_src1, a_dst1])
    av2 = jnp.stack([a_src2, a_dst2])

    h1, alpha1, cm1 = _tc_first(xp, W1, b1[None, :], av1)
    aggp1 = _sc_edge(h1, alpha1, cm1, edges)
    h2, alpha2, cm2 = _tc_mid(aggp1, W2, b2[None, :], av2)
    aggp2 = _sc_edge(h2, alpha2, cm2, edges)
    return _tc_final(aggp2)


# speedup vs baseline: 14.9287x; 1.0697x over previous
import functools

import jax
import jax.numpy as jnp
from jax import lax
from jax.experimental import pallas as pl
from jax.experimental.pallas import tpu as pltpu
from jax.experimental.pallas import tpu_sc as plsc

N = 10000
D = 128
E = 320000
MAXN = 1.0 - 1e-5

NC = 2
NS = 16
CH = 128
DH = D // NC
NP = 10240
EP = 327680
TPE = EP // NS
CPT = TPE // CH
NPT = NP // NS

RBH = 512
GRIDH = NP // RBH
RB = 400
GRID = N // RB

f32 = jnp.float32
i32 = jnp.int32


def _u_scale(n):
    nu = jnp.minimum(jnp.tanh(n), MAXN)
    nu = jnp.maximum(nu, 1e-15)
    at = 0.5 * jnp.log((1.0 + nu) / (1.0 - nu))
    return at / n


def _mobius_attention_head(u, W_ref, b_ref, av_ref, h_ref, al_ref, cm_ref):
    h = jnp.dot(u, W_ref[...], preferred_element_type=f32) + b_ref[...]
    h_ref[0, :, :] = h[:, 0:DH]
    h_ref[1, :, :] = h[:, DH:D]
    i = pl.program_id(0)
    sl = pl.ds(i * RBH, RBH)
    asrc = jnp.sum(h * av_ref[0:1, :], axis=1)
    al_ref[0:1, sl] = asrc[None, :]
    al_ref[1:2, sl] = jnp.sum(h * av_ref[1:2, :], axis=1)[None, :]
    blkmax = jnp.full((1, D), jnp.max(asrc), f32)

    @pl.when(i == 0)
    def _():
        cm_ref[...] = blkmax

    cm_ref[...] = jnp.maximum(cm_ref[...], blkmax)


def _tc_first_body(x_ref, W_ref, b_ref, av_ref, h_ref, al_ref, cm_ref):
    xb = x_ref[...]
    n = jnp.maximum(jnp.sqrt(jnp.sum(xb * xb, axis=1, keepdims=True)), 1e-15)
    u = xb * _u_scale(n)
    _mobius_attention_head(u, W_ref, b_ref, av_ref, h_ref, al_ref, cm_ref)


def _tc_mid_body(agg_ref, W_ref, b_ref, av_ref, h_ref, al_ref, cm_ref):
    g = jnp.maximum(jnp.concatenate([agg_ref[0], agg_ref[1]], axis=1), 0.0)
    n = jnp.maximum(jnp.sqrt(jnp.sum(g * g, axis=1, keepdims=True)), 1e-15)
    u = g * _u_scale(n)
    _mobius_attention_head(u, W_ref, b_ref, av_ref, h_ref, al_ref, cm_ref)


def _tc_final_body(agg_ref, y_ref):
    g = jnp.maximum(jnp.concatenate([agg_ref[0], agg_ref[1]], axis=1), 0.0)
    n = jnp.maximum(jnp.sqrt(jnp.sum(g * g, axis=1, keepdims=True)), 1e-15)
    nu = jnp.minimum(jnp.tanh(n), MAXN)
    y_ref[...] = g * (nu / n)


_mat_specs = [
    pl.BlockSpec((D, D), lambda i: (0, 0)),
    pl.BlockSpec((1, D), lambda i: (0, 0)),
    pl.BlockSpec((2, D), lambda i: (0, 0)),
]
_head_out_shape = [
    jax.ShapeDtypeStruct((NC, NP, DH), f32),
    jax.ShapeDtypeStruct((2, NP), f32),
    jax.ShapeDtypeStruct((1, D), f32),
]
_head_out_specs = [
    pl.BlockSpec((NC, RBH, DH), lambda i: (0, i, 0)),
    pl.BlockSpec((2, NP), lambda i: (0, 0)),
    pl.BlockSpec((1, D), lambda i: (0, 0)),
]


def _tc_first(x, W, b2, av):
    return pl.pallas_call(
        _tc_first_body,
        grid=(GRIDH,),
        in_specs=[pl.BlockSpec((RBH, D), lambda i: (i, 0))] + _mat_specs,
        out_specs=_head_out_specs,
        out_shape=_head_out_shape,
    )(x, W, b2, av)


def _tc_mid(aggp, W, b2, av):
    return pl.pallas_call(
        _tc_mid_body,
        grid=(GRIDH,),
        in_specs=[pl.BlockSpec((NC, RBH, DH), lambda i: (0, i, 0))] + _mat_specs,
        out_specs=_head_out_specs,
        out_shape=_head_out_shape,
    )(aggp, W, b2, av)


def _tc_final(aggp):
    return pl.pallas_call(
        _tc_final_body,
        grid=(GRID,),
        in_specs=[pl.BlockSpec((NC, RB, DH), lambda i: (0, i, 0))],
        out_specs=pl.BlockSpec((RB, D), lambda i: (i, 0)),
        out_shape=jax.ShapeDtypeStruct((N, D), f32),
    )(aggp)


def _sc_body(h_hbm, alpha_hbm, cm_hbm, edge_hbm, out_hbm,
             asrc_v, adst_v, denom_v, rbuf, wbig,
             attnbuf, cbuf, zbuf, rows, sem, sem2, denom_sp, agg_sp):
    c = lax.axis_index("c")
    s = lax.axis_index("s")
    zeros16 = jnp.zeros((16,), f32)

    pltpu.sync_copy(cm_hbm.at[0], cbuf)

    pltpu.sync_copy(alpha_hbm.at[0], asrc_v)
    pltpu.sync_copy(alpha_hbm.at[1], adst_v)

    def _zrows(i, carry):
        for k in range(DH // 16):
            rows[0, i, pl.ds(k * 16, 16)] = zeros16
        return carry

    lax.fori_loop(0, CH, _zrows, 0)
    for k in range(CH // 16):
        zbuf[pl.ds(k * 16, 16)] = zeros16

    nbase = s * NPT
    for k in range(NPT // CH):
        pltpu.sync_copy(rows.at[0], agg_sp.at[pl.ds(nbase + k * CH, CH), :])
        pltpu.sync_copy(zbuf, denom_sp.at[pl.ds(nbase + k * CH, CH)])
    plsc.subcore_barrier()

    C16 = jnp.maximum(cbuf[pl.ds(0, 16)], 0.0)

    base_e = s * TPE

    def _edge_w(sb, db, k):
        sv = sb[pl.ds(k * 16, 16)]
        dv = db[pl.ds(k * 16, 16)]
        a_s = plsc.load_gather(asrc_v, [sv])
        a_d = plsc.load_gather(adst_v, [dv])
        t = a_s + a_d
        e = jnp.maximum(t, 0.2 * t)
        m0 = C16 + a_d
        M = jnp.maximum(m0, 0.2 * m0)
        return jnp.exp(e - M)

    def _estage(j, b):
        off = base_e + j * CH
        return pltpu.make_async_copy(
            edge_hbm.at[:, pl.ds(off, CH)], rbuf.at[b], sem
        )

    _estage(0, 0).start()

    def _scalar(jo, carry):
        for b in range(2):
            j = jo * 2 + b
            nb = (b + 1) % 2
            _estage(j, b).wait()

            @pl.when(j + 1 < CPT)
            def _():
                @pl.when(j >= 1)
                def _():
                    pltpu.make_async_copy(
                        wbig.at[j - 1], denom_sp.at[rbuf.at[nb, 1]], sem2
                    ).wait()

                _estage(j + 1, nb).start()

            sb, db, wb = rbuf.at[b, 0], rbuf.at[b, 1], wbig.at[j]
            for k in range(CH // 16):
                wb[pl.ds(k * 16, 16)] = _edge_w(sb, db, k)
            pltpu.async_copy(wb, denom_sp.at[db], sem2, add=True)
        return carry

    lax.fori_loop(0, CPT // 2, _scalar, 0)
    pltpu.make_async_copy(
        wbig.at[CPT - 2], denom_sp.at[rbuf.at[(CPT - 2) % 2, 1]], sem2
    ).wait()
    pltpu.make_async_copy(
        wbig.at[CPT - 1], denom_sp.at[rbuf.at[(CPT - 1) % 2, 1]], sem2
    ).wait()
    plsc.subcore_barrier()

    pltpu.sync_copy(denom_sp, denom_v)

    hc = h_hbm.at[c]

    def _stage(j, b):
        off = base_e + j * CH
        pltpu.sync_copy(edge_hbm.at[:, pl.ds(off, CH)], rbuf.at[b])

    _stage(0, 0)
    pltpu.async_copy(hc.at[rbuf.at[0, 0]], rows.at[0], sem)

    def _rowchunk(jo, carry):
        for b in range(2):
            j = jo * 2 + b
            nb = (b + 1) % 2
            sb, db = rbuf.at[b, 0], rbuf.at[b, 1]
            pltpu.make_async_copy(hc.at[sb], rows.at[b], sem).wait()

            @pl.when(j + 1 < CPT)
            def _():
                @pl.when(j >= 1)
                def _():
                    pltpu.make_async_copy(
                        rows.at[nb], agg_sp.at[rbuf.at[nb, 1]], sem2
                    ).wait()

                _stage(j + 1, nb)
                pltpu.async_copy(hc.at[rbuf.at[nb, 0]], rows.at[nb], sem)

            for k in range(CH // 16):
                wv = wbig[j, pl.ds(k * 16, 16)]
                dv = db[pl.ds(k * 16, 16)]
                dn = plsc.load_gather(denom_v, [dv])
                attnbuf[pl.ds(k * 16, 16)] = wv / (dn + 1e-15)

            def _srow(ro, carry2):
                for u in range(4):
                    r = ro * 4 + u
                    av = plsc.load_gather(attnbuf, [jnp.full((16,), r, i32)])
                    for k in range(DH // 16):
                        sl = pl.ds(k * 16, 16)
                        rows[b, r, sl] = rows[b, r, sl] * av
                return carry2

            lax.fori_loop(0, CH // 4, _srow, 0)
            pltpu.async_copy(rows.at[b], agg_sp.at[db], add=True, sem=sem2)
        return carry

    lax.fori_loop(0, CPT // 2, _rowchunk, 0)
    pltpu.make_async_copy(
        rows.at[(CPT - 2) % 2], agg_sp.at[rbuf.at[(CPT - 2) % 2, 1]], sem2
    ).wait()
    pltpu.make_async_copy(
        rows.at[(CPT - 1) % 2], agg_sp.at[rbuf.at[(CPT - 1) % 2, 1]], sem2
    ).wait()
    plsc.subcore_barrier()

    for k in range(NPT // CH):
        sl = pl.ds(nbase + k * CH, CH)
        pltpu.sync_copy(agg_sp.at[sl, :], out_hbm.at[c, sl, :])


@functools.cache
def _sc_edge_kernel():
    return pl.kernel(
        _sc_body,
        out_type=jax.ShapeDtypeStruct((NC, NP, DH), f32),
        mesh=plsc.VectorSubcoreMesh(
            core_axis_name="c", subcore_axis_name="s",
            num_cores=NC, num_subcores=NS,
        ),
        compiler_params=pltpu.CompilerParams(
            needs_layout_passes=False, use_tc_tiling_on_sc=False
        ),
        scratch_types=[
            pltpu.VMEM((NP,), f32),
            pltpu.VMEM((NP,), f32),
            pltpu.VMEM((NP,), f32),
            pltpu.VMEM((2, 2, CH), i32),
            pltpu.VMEM((CPT, CH), f32),
            pltpu.VMEM((CH,), f32),
            pltpu.VMEM((D,), f32),
            pltpu.VMEM((CH,), f32),
            pltpu.VMEM((2, CH, DH), f32),
            pltpu.SemaphoreType.DMA,
            pltpu.SemaphoreType.DMA,
            pltpu.VMEM_SHARED((NP,), f32),
            pltpu.VMEM_SHARED((NP, DH), f32),
        ],
    )


def _sc_edge(h, alpha, cm, edges):
    return _sc_edge_kernel()(h, alpha, cm, edges)


def kernel(x, edge_index, W1, b1, a_src1, a_dst1, W2, b2, a_src2, a_dst2):
    pad = EP - E
    srcp = jnp.concatenate([edge_index[0], jnp.zeros((pad,), i32)])
    dstp = jnp.concatenate([edge_index[1], jnp.full((pad,), NP - 1, i32)])
    edges = jnp.stack([srcp, dstp])
    xp = jnp.concatenate([x, jnp.zeros((NP - N, D), f32)])

    av1 = jnp.stack([a_src1, a_dst1])
    av2 = jnp.stack([a_src2, a_dst2])

    h1, alpha1, cm1 = _tc_first(xp, W1, b1[None, :], av1)
    aggp1 = _sc_edge(h1, alpha1, cm1, edges)
    h2, alpha2, cm2 = _tc_mid(aggp1, W2, b2[None, :], av2)
    aggp2 = _sc_edge(h2, alpha2, cm2, edges)
    return _tc_final(aggp2)
